# baseline (speedup 1.0000x reference)
import functools

import jax
import jax.numpy as jnp
from jax import lax
from jax.experimental import pallas as pl
from jax.experimental.pallas import tpu as pltpu
from jax.experimental.pallas import tpu_sc as plsc

NUM_TIME_BINS = 1000
MAX_TIME_DIFF = 26000000.0
D_NODE, D_EDGE, D_TIME = 256, 128, 128
NBR = 32
LANES = 16
NW = 32
N_NODE_T = 16
N_EDGE_T = 8
PAIR = 2
NBUF = 2
GRP = 16
CURC = 16
JU = 8


def _sc_gather_agg(node_feats, edge_feats, lpe_ext, qids, nbr_flat, eid_flat,
                   tt_pack):
  R = qids.shape[0]
  RN = R // N_NODE_T
  RE = R // N_EDGE_T
  PPG = GRP // PAIR
  mesh = plsc.VectorSubcoreMesh(core_axis_name="c", subcore_axis_name="s")

  @functools.partial(
      pl.kernel,
      mesh=mesh,
      out_type=(
          jax.ShapeDtypeStruct((R, D_NODE), jnp.float32),
          jax.ShapeDtypeStruct((R, D_EDGE), jnp.float32),
          jax.ShapeDtypeStruct((R, D_TIME), jnp.float32),
          jax.ShapeDtypeStruct((R, D_NODE), jnp.float32),
      ),
      scratch_types=[
          pltpu.VMEM((RN,), jnp.int32),
          pltpu.VMEM((4 * GRP * NBR,), jnp.int32),
          pltpu.VMEM((4 * GRP, 2 * NBR), jnp.float32),
          pltpu.VMEM((3 * GRP * NBR,), jnp.int32),
          pltpu.VMEM((NBUF * PAIR * NBR, D_NODE), jnp.float32),
          pltpu.VMEM((NBUF * PAIR * NBR, D_EDGE), jnp.float32),
          pltpu.VMEM((2 * CURC, D_NODE), jnp.float32),
          pltpu.VMEM((GRP, D_NODE), jnp.float32),
          pltpu.VMEM((GRP, D_EDGE), jnp.float32),
          pltpu.VMEM_SHARED((NUM_TIME_BINS + 2, D_TIME), jnp.float32),
          pltpu.SemaphoreType.DMA,
          pltpu.SemaphoreType.DMA,
          pltpu.SemaphoreType.DMA,
          pltpu.SemaphoreType.DMA,
          pltpu.SemaphoreType.DMA,
      ],
  )
  def k(node_hbm, edge_hbm, lpe_hbm, qid_hbm, nbr_hbm, eid_hbm, tt_hbm,
        aggn_hbm, agge_hbm, aggl_hbm, cur_hbm,
        qid_v, idb, ttb, binsb, nring, ering, curbuf, outbN, outbE,
        lpe_sh, rs0, rs1, cs0, cs1, isem):
    wid = lax.axis_index("s") * 2 + lax.axis_index("c")
    rsem = [rs0, rs1]
    csem = [cs0, cs1]
    inv = 1.0 / NBR

    @pl.when(lax.axis_index("s") == 0)
    def _():
      pltpu.sync_copy(lpe_hbm, lpe_sh)
    plsc.subcore_barrier()

    def class_prog(base, rows, id_hbm, src_hbm, ring, width, outb, out_hbm,
                   with_bins):
      G = rows // GRP

      def load_ids(g, slot, sync):
        src = id_hbm.at[pl.ds((base + g * GRP) * NBR, GRP * NBR)]
        dst = idb.at[pl.ds(slot * GRP * NBR, GRP * NBR)]
        cps = [(src, dst)]
        if with_bins:
          cps.append((tt_hbm.at[pl.ds(base + g * GRP, GRP)],
                      ttb.at[pl.ds(slot * GRP, GRP)]))
        for s_, d_ in cps:
          if sync:
            pltpu.sync_copy(s_, d_)
          else:
            pltpu.async_copy(s_, d_, isem)

      def wait_ids():
        pltpu.make_async_copy(
            id_hbm.at[pl.ds(0, GRP * NBR)],
            idb.at[pl.ds(0, GRP * NBR)], isem).wait()
        if with_bins:
          pltpu.make_async_copy(
              tt_hbm.at[pl.ds(0, GRP)], ttb.at[pl.ds(0, GRP)], isem).wait()

      def compute_bins(gg):
        pids = lax.rem(gg, 4) * GRP
        pb = lax.rem(gg, 3) * GRP * NBR

        def bins_row(r, rc):
          for h in range(NBR // LANES):
            t_vec = ttb[pids + r, pl.ds(h * LANES, LANES)]
            nt_vec = ttb[pids + r, pl.ds(NBR + h * LANES, LANES)]
            td = t_vec - nt_vec
            clamped = jnp.minimum(jnp.maximum(td, 0.0), MAX_TIME_DIFF)
            normalized = clamped / MAX_TIME_DIFF
            b = (normalized * float(NUM_TIME_BINS)).astype(jnp.int32)
            b = jnp.minimum(b, NUM_TIME_BINS)
            nbr_vec = idb[pl.ds((pids + r) * NBR + h * LANES, LANES)]
            b = jnp.where(nbr_vec == 0, NUM_TIME_BINS + 1, b)
            binsb[pl.ds(pb + r * NBR + h * LANES, LANES)] = b
          return rc

        lax.fori_loop(0, GRP, bins_row, 0)

      def fire_pair(tp, slot):
        tg = tp // PPG
        idx = lax.rem(tp, PPG)
        rows_sl = pl.ds(slot * PAIR * NBR, PAIR * NBR)
        if with_bins:
          off = lax.rem(tg, 3) * GRP * NBR + idx * PAIR * NBR
          isrc = binsb.at[pl.ds(off, PAIR * NBR)]
        else:
          off = lax.rem(tg, 4) * GRP * NBR + idx * PAIR * NBR
          isrc = idb.at[pl.ds(off, PAIR * NBR)]
        pltpu.async_copy(src_hbm.at[isrc], ring.at[rows_sl], rsem[slot])

      def wait_slot(slot):
        rows_sl = pl.ds(slot * PAIR * NBR, PAIR * NBR)
        pltpu.make_async_copy(
            src_hbm.at[pl.ds(0, PAIR * NBR)], ring.at[rows_sl],
            rsem[slot]).wait()

      load_ids(0, 0, sync=True)
      if with_bins:
        compute_bins(0)
      load_ids(1, 1, sync=True)
      if with_bins:
        compute_bins(1)
      load_ids(2, 2, sync=False)
      for b in range(NBUF):
        fire_pair(b, b)

      def g_body(g, carry):
        @pl.when(g + 2 < G)
        def _():
          wait_ids()
          if with_bins:
            compute_bins(g + 2)

        @pl.when(g + 3 < G)
        def _():
          load_ids(g + 3, lax.rem(g + 3, 4), sync=False)

        def step_body(s, sc):
          for b in range(NBUF):
            tp = g * PPG + s * NBUF + b
            wait_slot(b)
            for pr in range(PAIR):
              orow = (s * NBUF + b) * PAIR + pr

              def jbody(jc, accs, b=b, pr=pr):
                out = list(accs)
                for jj in range(JU):
                  row = (b * PAIR + pr) * NBR + jc * JU + jj
                  for d in range(width):
                    out[d] = out[d] + ring[row, pl.ds(d * LANES, LANES)]
                return tuple(out)

              zero = jnp.zeros((LANES,), jnp.float32)
              accs = lax.fori_loop(0, NBR // JU, jbody, (zero,) * width)
              for d in range(width):
                outb[orow, pl.ds(d * LANES, LANES)] = accs[d] * inv

            @pl.when(tp < rows // PAIR - NBUF)
            def _():
              fire_pair(tp + NBUF, b)

          return sc

        lax.fori_loop(0, PPG // NBUF, step_body, carry)
        pltpu.sync_copy(outb, out_hbm.at[pl.ds(base + g * GRP, GRP)])
        return carry

      lax.fori_loop(0, G, g_body, 0)

    @pl.when(wid < N_NODE_T)
    def _():
      nbase = wid * RN
      pltpu.sync_copy(qid_hbm.at[pl.ds(nbase, RN)], qid_v)

      def fire_cur(g, p):
        return pltpu.async_copy(
            node_hbm.at[qid_v.at[pl.ds(g * CURC, CURC)]],
            curbuf.at[pl.ds(p * CURC, CURC)], csem[p])

      hs = {0: fire_cur(0, 0)}
      for g in range(RN // CURC):
        p = g % 2
        if g + 1 < RN // CURC:
          hs[g + 1] = fire_cur(g + 1, (g + 1) % 2)
        hs[g].wait()
        pltpu.sync_copy(curbuf.at[pl.ds(p * CURC, CURC)],
                        cur_hbm.at[pl.ds(nbase + g * CURC, CURC)])

      with jax.named_scope("nodeclass"):
        class_prog(nbase, RN, nbr_hbm, node_hbm, nring, D_NODE // LANES,
                   outbN, aggn_hbm, with_bins=False)

    @pl.when((wid >= N_NODE_T) & (wid < N_NODE_T + N_EDGE_T))
    def _():
      with jax.named_scope("edgeclass"):
        class_prog((wid - N_NODE_T) * RE, RE, eid_hbm, edge_hbm, ering,
                   D_EDGE // LANES, outbE, agge_hbm, with_bins=False)

    @pl.when(wid >= N_NODE_T + N_EDGE_T)
    def _():
      with jax.named_scope("lpeclass"):
        class_prog((wid - N_NODE_T - N_EDGE_T) * RE, RE, nbr_hbm, lpe_sh,
                   ering, D_TIME // LANES, outbE, aggl_hbm, with_bins=True)

  return k(node_feats, edge_feats, lpe_ext, qids, nbr_flat, eid_flat, tt_pack)


def _tc_matmul_relu(cur, aggn, agge, aggl, wc, wn, we, wl, b):
  R = cur.shape[0]
  BM = 512

  def mm(cur_ref, n_ref, e_ref, l_ref, wc_ref, wn_ref, we_ref, wl_ref, b_ref,
         o_ref):
    y = jnp.dot(cur_ref[...], wc_ref[...], preferred_element_type=jnp.float32)
    y += jnp.dot(n_ref[...], wn_ref[...], preferred_element_type=jnp.float32)
    y += jnp.dot(e_ref[...], we_ref[...], preferred_element_type=jnp.float32)
    y += jnp.dot(l_ref[...], wl_ref[...], preferred_element_type=jnp.float32)
    o_ref[...] = jnp.maximum(y + b_ref[...], 0.0)

  return pl.pallas_call(
      mm,
      grid=(R // BM,),
      in_specs=[
          pl.BlockSpec((BM, D_NODE), lambda i: (i, 0)),
          pl.BlockSpec((BM, D_NODE), lambda i: (i, 0)),
          pl.BlockSpec((BM, D_EDGE), lambda i: (i, 0)),
          pl.BlockSpec((BM, D_TIME), lambda i: (i, 0)),
          pl.BlockSpec((D_NODE, D_NODE), lambda i: (0, 0)),
          pl.BlockSpec((D_NODE, D_NODE), lambda i: (0, 0)),
          pl.BlockSpec((D_EDGE, D_NODE), lambda i: (0, 0)),
          pl.BlockSpec((D_TIME, D_NODE), lambda i: (0, 0)),
          pl.BlockSpec((1, D_NODE), lambda i: (0, 0)),
      ],
      out_specs=pl.BlockSpec((BM, D_NODE), lambda i: (i, 0)),
      out_shape=jax.ShapeDtypeStruct((R, D_NODE), jnp.float32),
  )(cur, aggn, agge, aggl, wc, wn, we, wl, b)


def kernel(node_raw_features, edge_raw_features, lpe_table, W_out, b_out,
           src_node_ids, dst_node_ids, node_interact_times,
           src_neighbor_ids, dst_neighbor_ids, src_edge_ids, dst_edge_ids,
           src_neighbor_times, dst_neighbor_times):
  B = src_node_ids.shape[0]
  i32 = jnp.int32
  qids = jnp.concatenate([src_node_ids, dst_node_ids]).astype(i32)
  nbr = jnp.concatenate([src_neighbor_ids, dst_neighbor_ids]).astype(i32)
  eids = jnp.concatenate([src_edge_ids, dst_edge_ids]).astype(i32)
  ntimes = jnp.concatenate([src_neighbor_times, dst_neighbor_times])
  tb = jnp.broadcast_to(node_interact_times[:, None], (B, NBR))
  tmat = jnp.concatenate([tb, tb])
  tt_pack = jnp.concatenate([tmat, ntimes], axis=1)
  lpe_ext = jnp.concatenate(
      [lpe_table, jnp.zeros((1, D_TIME), jnp.float32)], axis=0)

  aggn, agge, aggl, cur = _sc_gather_agg(
      node_raw_features, edge_raw_features, lpe_ext, qids,
      nbr.reshape(-1), eids.reshape(-1), tt_pack)
  out = _tc_matmul_relu(
      cur, aggn, agge, aggl,
      W_out[:D_NODE], W_out[D_NODE:2 * D_NODE],
      W_out[2 * D_NODE:2 * D_NODE + D_EDGE], W_out[2 * D_NODE + D_EDGE:],
      b_out.reshape(1, D_NODE))
  src_emb, dst_emb = out[:B], out[B:]
  return (src_emb, dst_emb, jnp.zeros_like(src_emb))


# speedup vs baseline: 1.0859x; 1.0859x over previous
import functools

import jax
import jax.numpy as jnp
from jax import lax
from jax.experimental import pallas as pl
from jax.experimental.pallas import tpu as pltpu
from jax.experimental.pallas import tpu_sc as plsc

NUM_TIME_BINS = 1000
MAX_TIME_DIFF = 26000000.0
D_NODE, D_EDGE, D_TIME = 256, 128, 128
NBR = 32
LANES = 16
NW = 32
N_NODE_T = 16
N_EDGE_T = 9
PAIR = 2
NBUF = 2
GRP = 16
CURC = 16
JU = 8


def _sc_gather_agg(node_feats, edge_feats, lpe_ext, qids, nbr_flat, eid_flat,
                   tt_pack):
  R = qids.shape[0]
  RN = R // N_NODE_T
  RE = R // N_EDGE_T
  PPG = GRP // PAIR
  mesh = plsc.VectorSubcoreMesh(core_axis_name="c", subcore_axis_name="s")

  @functools.partial(
      pl.kernel,
      mesh=mesh,
      out_type=(
          jax.ShapeDtypeStruct((R, D_NODE), jnp.float32),
          jax.ShapeDtypeStruct((R, D_EDGE), jnp.float32),
          jax.ShapeDtypeStruct((R, D_TIME), jnp.float32),
          jax.ShapeDtypeStruct((R, D_NODE), jnp.float32),
      ),
      scratch_types=[
          pltpu.VMEM((RN,), jnp.int32),
          pltpu.VMEM((4 * GRP * NBR,), jnp.int32),
          pltpu.VMEM((4 * GRP, 2 * NBR), jnp.float32),
          pltpu.VMEM((3 * GRP * NBR,), jnp.int32),
          pltpu.VMEM((NBUF * PAIR * NBR, D_NODE), jnp.float32),
          pltpu.VMEM((NBUF * PAIR * NBR, D_EDGE), jnp.float32),
          pltpu.VMEM((2 * CURC, D_NODE), jnp.float32),
          pltpu.VMEM((GRP, D_NODE), jnp.float32),
          pltpu.VMEM((GRP, D_EDGE), jnp.float32),
          pltpu.VMEM_SHARED((NUM_TIME_BINS + 2, D_TIME), jnp.float32),
          pltpu.SemaphoreType.DMA,
          pltpu.SemaphoreType.DMA,
          pltpu.SemaphoreType.DMA,
          pltpu.SemaphoreType.DMA,
          pltpu.SemaphoreType.DMA,
      ],
  )
  def k(node_hbm, edge_hbm, lpe_hbm, qid_hbm, nbr_hbm, eid_hbm, tt_hbm,
        aggn_hbm, agge_hbm, aggl_hbm, cur_hbm,
        qid_v, idb, ttb, binsb, nring, ering, curbuf, outbN, outbE,
        lpe_sh, rs0, rs1, cs0, cs1, isem):
    wid = lax.axis_index("s") * 2 + lax.axis_index("c")
    rsem = [rs0, rs1]
    csem = [cs0, cs1]
    inv = 1.0 / NBR

    @pl.when(lax.axis_index("s") == 0)
    def _():
      pltpu.sync_copy(lpe_hbm, lpe_sh)
    plsc.subcore_barrier()

    def class_prog(base, rows, id_hbm, src_hbm, ring, width, outb, out_hbm,
                   with_bins):
      G = rows // GRP

      def load_ids(g, slot, sync):
        src = id_hbm.at[pl.ds((base + g * GRP) * NBR, GRP * NBR)]
        dst = idb.at[pl.ds(slot * GRP * NBR, GRP * NBR)]
        cps = [(src, dst)]
        if with_bins:
          cps.append((tt_hbm.at[pl.ds(base + g * GRP, GRP)],
                      ttb.at[pl.ds(slot * GRP, GRP)]))
        for s_, d_ in cps:
          if sync:
            pltpu.sync_copy(s_, d_)
          else:
            pltpu.async_copy(s_, d_, isem)

      def wait_ids():
        pltpu.make_async_copy(
            id_hbm.at[pl.ds(0, GRP * NBR)],
            idb.at[pl.ds(0, GRP * NBR)], isem).wait()
        if with_bins:
          pltpu.make_async_copy(
              tt_hbm.at[pl.ds(0, GRP)], ttb.at[pl.ds(0, GRP)], isem).wait()

      def compute_bins(gg):
        pids = lax.rem(gg, 4) * GRP
        pb = lax.rem(gg, 3) * GRP * NBR

        def bins_row(r, rc):
          for h in range(NBR // LANES):
            t_vec = ttb[pids + r, pl.ds(h * LANES, LANES)]
            nt_vec = ttb[pids + r, pl.ds(NBR + h * LANES, LANES)]
            td = t_vec - nt_vec
            clamped = jnp.minimum(jnp.maximum(td, 0.0), MAX_TIME_DIFF)
            normalized = clamped / MAX_TIME_DIFF
            b = (normalized * float(NUM_TIME_BINS)).astype(jnp.int32)
            b = jnp.minimum(b, NUM_TIME_BINS)
            nbr_vec = idb[pl.ds((pids + r) * NBR + h * LANES, LANES)]
            b = jnp.where(nbr_vec == 0, NUM_TIME_BINS + 1, b)
            binsb[pl.ds(pb + r * NBR + h * LANES, LANES)] = b
          return rc

        lax.fori_loop(0, GRP, bins_row, 0)

      def fire_pair(tp, slot):
        tg = tp // PPG
        idx = lax.rem(tp, PPG)
        rows_sl = pl.ds(slot * PAIR * NBR, PAIR * NBR)
        if with_bins:
          off = lax.rem(tg, 3) * GRP * NBR + idx * PAIR * NBR
          isrc = binsb.at[pl.ds(off, PAIR * NBR)]
        else:
          off = lax.rem(tg, 4) * GRP * NBR + idx * PAIR * NBR
          isrc = idb.at[pl.ds(off, PAIR * NBR)]
        pltpu.async_copy(src_hbm.at[isrc], ring.at[rows_sl], rsem[slot])

      def wait_slot(slot):
        rows_sl = pl.ds(slot * PAIR * NBR, PAIR * NBR)
        pltpu.make_async_copy(
            src_hbm.at[pl.ds(0, PAIR * NBR)], ring.at[rows_sl],
            rsem[slot]).wait()

      load_ids(0, 0, sync=True)
      if with_bins:
        compute_bins(0)
      load_ids(1, 1, sync=True)
      if with_bins:
        compute_bins(1)
      load_ids(2, 2, sync=False)
      for b in range(NBUF):
        fire_pair(b, b)

      def g_body(g, carry):
        @pl.when(g + 2 < G)
        def _():
          wait_ids()
          if with_bins:
            compute_bins(g + 2)

        @pl.when(g + 3 < G)
        def _():
          load_ids(g + 3, lax.rem(g + 3, 4), sync=False)

        def step_body(s, sc):
          for b in range(NBUF):
            tp = g * PPG + s * NBUF + b
            wait_slot(b)
            for pr in range(PAIR):
              orow = (s * NBUF + b) * PAIR + pr

              def jbody(jc, accs, b=b, pr=pr):
                out = list(accs)
                for jj in range(JU):
                  row = (b * PAIR + pr) * NBR + jc * JU + jj
                  for d in range(width):
                    out[d] = out[d] + ring[row, pl.ds(d * LANES, LANES)]
                return tuple(out)

              zero = jnp.zeros((LANES,), jnp.float32)
              accs = lax.fori_loop(0, NBR // JU, jbody, (zero,) * width)
              for d in range(width):
                outb[orow, pl.ds(d * LANES, LANES)] = accs[d] * inv

            @pl.when(tp < rows // PAIR - NBUF)
            def _():
              fire_pair(tp + NBUF, b)

          return sc

        lax.fori_loop(0, PPG // NBUF, step_body, carry)
        pltpu.sync_copy(outb, out_hbm.at[pl.ds(base + g * GRP, GRP)])
        return carry

      lax.fori_loop(0, G, g_body, 0)

    @pl.when(wid < N_NODE_T)
    def _():
      nbase = wid * RN
      pltpu.sync_copy(qid_hbm.at[pl.ds(nbase, RN)], qid_v)

      def fire_cur(g, p):
        return pltpu.async_copy(
            node_hbm.at[qid_v.at[pl.ds(g * CURC, CURC)]],
            curbuf.at[pl.ds(p * CURC, CURC)], csem[p])

      hs = {0: fire_cur(0, 0)}
      for g in range(RN // CURC):
        p = g % 2
        if g + 1 < RN // CURC:
          hs[g + 1] = fire_cur(g + 1, (g + 1) % 2)
        hs[g].wait()
        pltpu.sync_copy(curbuf.at[pl.ds(p * CURC, CURC)],
                        cur_hbm.at[pl.ds(nbase + g * CURC, CURC)])

      with jax.named_scope("nodeclass"):
        class_prog(nbase, RN, nbr_hbm, node_hbm, nring, D_NODE // LANES,
                   outbN, aggn_hbm, with_bins=False)

    @pl.when((wid >= N_NODE_T) & (wid < N_NODE_T + N_EDGE_T))
    def _():
      e = wid - N_NODE_T
      erows = jnp.where(e < 8, 912, 896)
      with jax.named_scope("edgeclass"):
        class_prog(e * 912, erows, eid_hbm, edge_hbm, ering,
                   D_EDGE // LANES, outbE, agge_hbm, with_bins=False)

    @pl.when(wid >= N_NODE_T + N_EDGE_T)
    def _():
      l = wid - N_NODE_T - N_EDGE_T
      lrows = jnp.where(l < 6, 1168, 1184)
      with jax.named_scope("lpeclass"):
        class_prog(l * 1168, lrows, nbr_hbm, lpe_sh,
                   ering, D_TIME // LANES, outbE, aggl_hbm, with_bins=True)

  return k(node_feats, edge_feats, lpe_ext, qids, nbr_flat, eid_flat, tt_pack)


def _tc_matmul_relu(cur, aggn, agge, aggl, wc, wn, we, wl, b):
  R = cur.shape[0]
  BM = 512

  def mm(cur_ref, n_ref, e_ref, l_ref, wc_ref, wn_ref, we_ref, wl_ref, b_ref,
         o_ref):
    y = jnp.dot(cur_ref[...], wc_ref[...], preferred_element_type=jnp.float32)
    y += jnp.dot(n_ref[...], wn_ref[...], preferred_element_type=jnp.float32)
    y += jnp.dot(e_ref[...], we_ref[...], preferred_element_type=jnp.float32)
    y += jnp.dot(l_ref[...], wl_ref[...], preferred_element_type=jnp.float32)
    o_ref[...] = jnp.maximum(y + b_ref[...], 0.0)

  return pl.pallas_call(
      mm,
      grid=(R // BM,),
      in_specs=[
          pl.BlockSpec((BM, D_NODE), lambda i: (i, 0)),
          pl.BlockSpec((BM, D_NODE), lambda i: (i, 0)),
          pl.BlockSpec((BM, D_EDGE), lambda i: (i, 0)),
          pl.BlockSpec((BM, D_TIME), lambda i: (i, 0)),
          pl.BlockSpec((D_NODE, D_NODE), lambda i: (0, 0)),
          pl.BlockSpec((D_NODE, D_NODE), lambda i: (0, 0)),
          pl.BlockSpec((D_EDGE, D_NODE), lambda i: (0, 0)),
          pl.BlockSpec((D_TIME, D_NODE), lambda i: (0, 0)),
          pl.BlockSpec((1, D_NODE), lambda i: (0, 0)),
      ],
      out_specs=pl.BlockSpec((BM, D_NODE), lambda i: (i, 0)),
      out_shape=jax.ShapeDtypeStruct((R, D_NODE), jnp.float32),
  )(cur, aggn, agge, aggl, wc, wn, we, wl, b)


def kernel(node_raw_features, edge_raw_features, lpe_table, W_out, b_out,
           src_node_ids, dst_node_ids, node_interact_times,
           src_neighbor_ids, dst_neighbor_ids, src_edge_ids, dst_edge_ids,
           src_neighbor_times, dst_neighbor_times):
  B = src_node_ids.shape[0]
  i32 = jnp.int32
  qids = jnp.concatenate([src_node_ids, dst_node_ids]).astype(i32)
  nbr = jnp.concatenate([src_neighbor_ids, dst_neighbor_ids]).astype(i32)
  eids = jnp.concatenate([src_edge_ids, dst_edge_ids]).astype(i32)
  ntimes = jnp.concatenate([src_neighbor_times, dst_neighbor_times])
  tb = jnp.broadcast_to(node_interact_times[:, None], (B, NBR))
  tmat = jnp.concatenate([tb, tb])
  tt_pack = jnp.concatenate([tmat, ntimes], axis=1)
  lpe_ext = jnp.concatenate(
      [lpe_table, jnp.zeros((1, D_TIME), jnp.float32)], axis=0)

  aggn, agge, aggl, cur = _sc_gather_agg(
      node_raw_features, edge_raw_features, lpe_ext, qids,
      nbr.reshape(-1), eids.reshape(-1), tt_pack)
  out = _tc_matmul_relu(
      cur, aggn, agge, aggl,
      W_out[:D_NODE], W_out[D_NODE:2 * D_NODE],
      W_out[2 * D_NODE:2 * D_NODE + D_EDGE], W_out[2 * D_NODE + D_EDGE:],
      b_out.reshape(1, D_NODE))
  src_emb, dst_emb = out[:B], out[B:]
  return (src_emb, dst_emb, jnp.zeros_like(src_emb))
